# baseline (speedup 1.0000x reference)
import functools

import jax
import jax.numpy as jnp
from jax.experimental import pallas as pl
from jax.experimental.pallas import tpu as pltpu

H = 8
D = 2048
DH = 8192
S = 2048
EPS = 1e-5
TCHUNK = 256
DHB = 512
NDH = DH // DHB


def _ffn_kernel(st_ref, nt_ref, off_ref,
                xs_ref, w1_ref, w2_ref, b1_ref, b2_ref, gamma_ref, beta_ref,
                out_ref):
    e = pl.program_id(0)
    dh = pl.program_id(1)

    @pl.when((e == 0) & (dh == 0))
    def _zero():
        out_ref[...] = jnp.zeros_like(out_ref)

    off_e = off_ref[e]
    off_e1 = off_ref[e + 1]
    st0 = st_ref[e]
    nt = nt_ref[e]

    def chunk_acc(i, _):
        w0 = st0 + i * TCHUNK
        r0 = pl.multiple_of(jnp.minimum(w0, S - TCHUNK), 8)
        lo = jnp.maximum(off_e, w0)
        hi = jnp.minimum(off_e1, w0 + TCHUNK)
        b1 = b1_ref[pl.ds(e, 1), pl.ds(dh * DHB, DHB)]
        HALF = TCHUNK // 2
        for half in range(2):
            rh = r0 + half * HALF
            xb = xs_ref[pl.ds(rh, HALF), :]
            h = jnp.dot(xb, w1_ref[0], preferred_element_type=jnp.float32,
                        precision=jax.lax.Precision.DEFAULT) + b1
            h = jnp.maximum(h, 0.0)
            gi = rh + jax.lax.broadcasted_iota(jnp.int32, (HALF, 1), 0)
            rowmask = (gi >= lo) & (gi < hi)
            h = jnp.where(rowmask, h, 0.0)
            y = jnp.dot(h, w2_ref[0], preferred_element_type=jnp.float32,
                        precision=jax.lax.Precision.DEFAULT)
            out_ref[pl.ds(rh, HALF), :] = out_ref[pl.ds(rh, HALF), :] + y
        return 0

    jax.lax.fori_loop(0, nt, chunk_acc, 0)

    @pl.when(dh == NDH - 1)
    def _finalize():
        b2 = b2_ref[pl.ds(e, 1), :]

        def chunk_fin(i, _):
            w0 = st0 + i * TCHUNK
            r0 = pl.multiple_of(jnp.minimum(w0, S - TCHUNK), 8)
            lo = jnp.maximum(off_e, w0)
            hi = jnp.minimum(off_e1, w0 + TCHUNK)
            acc = out_ref[pl.ds(r0, TCHUNK), :]
            xc = xs_ref[pl.ds(r0, TCHUNK), :]
            yres = acc + xc + b2
            mu = jnp.mean(yres, axis=-1, keepdims=True)
            var = jnp.mean((yres - mu) ** 2, axis=-1, keepdims=True)
            nrm = ((yres - mu) * jax.lax.rsqrt(var + EPS) * gamma_ref[...]
                   + beta_ref[...])
            gi = r0 + jax.lax.broadcasted_iota(jnp.int32, (TCHUNK, 1), 0)
            mask = (gi >= lo) & (gi < hi)
            out_ref[pl.ds(r0, TCHUNK), :] = jnp.where(mask, nrm, acc)
            return 0

        jax.lax.fori_loop(0, nt, chunk_fin, 0)


@functools.partial(jax.jit, static_argnames=())
def _grouped_ffn(st, nt, off, xs, W1b, W2b, b1, b2, gamma, beta):
    grid_spec = pltpu.PrefetchScalarGridSpec(
        num_scalar_prefetch=3,
        grid=(H, NDH),
        in_specs=[
            pl.BlockSpec((S, D), lambda e, dh, *_: (0, 0)),
            pl.BlockSpec((1, D, DHB), lambda e, dh, *_: (e, 0, dh)),
            pl.BlockSpec((1, DHB, D), lambda e, dh, *_: (e, dh, 0)),
            pl.BlockSpec((H, DH), lambda e, dh, *_: (0, 0)),
            pl.BlockSpec((H, D), lambda e, dh, *_: (0, 0)),
            pl.BlockSpec((1, D), lambda e, dh, *_: (0, 0)),
            pl.BlockSpec((1, D), lambda e, dh, *_: (0, 0)),
        ],
        out_specs=pl.BlockSpec((S, D), lambda e, dh, *_: (0, 0)),
    )
    return pl.pallas_call(
        _ffn_kernel,
        grid_spec=grid_spec,
        out_shape=jax.ShapeDtypeStruct((S, D), jnp.float32),
        compiler_params=pltpu.CompilerParams(
            dimension_semantics=("arbitrary", "arbitrary")),
    )(st, nt, off, xs, W1b, W2b, b1, b2, gamma, beta)


def kernel(x, W1, b1, W2, b2, gamma, beta, orig_input, hash_bin_map):
    flat = x.reshape(S, D)
    bins = jnp.take(hash_bin_map, orig_input.reshape(-1), axis=0)
    oh = (bins[:, None] == jnp.arange(H, dtype=bins.dtype)[None, :])
    oh = oh.astype(jnp.int32)
    csum = jnp.cumsum(oh, axis=0)
    rank = jnp.sum((csum - oh) * oh, axis=1)
    counts = csum[-1]
    off = jnp.concatenate([jnp.zeros((1,), jnp.int32),
                           jnp.cumsum(counts)]).astype(jnp.int32)
    pos = (jnp.take(off, bins) + rank).astype(jnp.int32)
    sort_idx = jnp.zeros((S,), jnp.int32).at[pos].set(
        jnp.arange(S, dtype=jnp.int32))
    xs = jnp.take(flat, sort_idx, axis=0)
    st = ((off[:H] // 8) * 8).astype(jnp.int32)
    nt = jnp.where(counts > 0,
                   (off[1:] - st + TCHUNK - 1) // TCHUNK,
                   0).astype(jnp.int32)

    out_sorted = _grouped_ffn(
        st, nt, off, xs, W1, W2,
        b1, b2, gamma.reshape(1, D), beta.reshape(1, D))

    out = jnp.take(out_sorted, pos, axis=0)
    return out.reshape(x.shape)


# speedup vs baseline: 1.0961x; 1.0961x over previous
import functools

import jax
import jax.numpy as jnp
from jax.experimental import pallas as pl
from jax.experimental.pallas import tpu as pltpu

H = 8
D = 2048
DH = 8192
S = 2048
EPS = 1e-5
TCHUNK = 512
DHB = 1024
NDH = DH // DHB


def _ffn_kernel(st_ref, nt_ref, off_ref,
                xs_ref, w1_ref, w2_ref, b1_ref, b2_ref, gamma_ref, beta_ref,
                out_ref):
    e = pl.program_id(0)
    dh = pl.program_id(1)

    @pl.when((e == 0) & (dh == 0))
    def _zero():
        out_ref[...] = jnp.zeros_like(out_ref)

    off_e = off_ref[e]
    off_e1 = off_ref[e + 1]
    st0 = st_ref[e]
    nt = nt_ref[e]

    def chunk_acc(i, _):
        w0 = st0 + i * TCHUNK
        r0 = pl.multiple_of(jnp.minimum(w0, S - TCHUNK), 8)
        lo = jnp.maximum(off_e, w0)
        hi = jnp.minimum(off_e1, w0 + TCHUNK)
        b1 = b1_ref[pl.ds(e, 1), pl.ds(dh * DHB, DHB)]
        HALF = TCHUNK // 2
        for half in range(2):
            rh = r0 + half * HALF
            xb = xs_ref[pl.ds(rh, HALF), :]
            h = jnp.dot(xb, w1_ref[0], preferred_element_type=jnp.float32,
                        precision=jax.lax.Precision.DEFAULT) + b1
            h = jnp.maximum(h, 0.0)
            gi = rh + jax.lax.broadcasted_iota(jnp.int32, (HALF, 1), 0)
            rowmask = (gi >= lo) & (gi < hi)
            h = jnp.where(rowmask, h, 0.0)
            y = jnp.dot(h, w2_ref[0], preferred_element_type=jnp.float32,
                        precision=jax.lax.Precision.DEFAULT)
            out_ref[pl.ds(rh, HALF), :] = out_ref[pl.ds(rh, HALF), :] + y
        return 0

    jax.lax.fori_loop(0, nt, chunk_acc, 0)

    @pl.when(dh == NDH - 1)
    def _finalize():
        b2 = b2_ref[pl.ds(e, 1), :]

        def chunk_fin(i, _):
            w0 = st0 + i * TCHUNK
            r0 = pl.multiple_of(jnp.minimum(w0, S - TCHUNK), 8)
            lo = jnp.maximum(off_e, w0)
            hi = jnp.minimum(off_e1, w0 + TCHUNK)
            acc = out_ref[pl.ds(r0, TCHUNK), :]
            xc = xs_ref[pl.ds(r0, TCHUNK), :]
            yres = acc + xc + b2
            mu = jnp.mean(yres, axis=-1, keepdims=True)
            var = jnp.mean((yres - mu) ** 2, axis=-1, keepdims=True)
            nrm = ((yres - mu) * jax.lax.rsqrt(var + EPS) * gamma_ref[...]
                   + beta_ref[...])
            gi = r0 + jax.lax.broadcasted_iota(jnp.int32, (TCHUNK, 1), 0)
            mask = (gi >= lo) & (gi < hi)
            out_ref[pl.ds(r0, TCHUNK), :] = jnp.where(mask, nrm, acc)
            return 0

        jax.lax.fori_loop(0, nt, chunk_fin, 0)


@functools.partial(jax.jit, static_argnames=())
def _grouped_ffn(st, nt, off, xs, W1b, W2b, b1, b2, gamma, beta):
    grid_spec = pltpu.PrefetchScalarGridSpec(
        num_scalar_prefetch=3,
        grid=(H, NDH),
        in_specs=[
            pl.BlockSpec((S, D), lambda e, dh, *_: (0, 0)),
            pl.BlockSpec((1, D, DHB), lambda e, dh, *_: (e, 0, dh)),
            pl.BlockSpec((1, DHB, D), lambda e, dh, *_: (e, dh, 0)),
            pl.BlockSpec((H, DH), lambda e, dh, *_: (0, 0)),
            pl.BlockSpec((H, D), lambda e, dh, *_: (0, 0)),
            pl.BlockSpec((1, D), lambda e, dh, *_: (0, 0)),
            pl.BlockSpec((1, D), lambda e, dh, *_: (0, 0)),
        ],
        out_specs=pl.BlockSpec((S, D), lambda e, dh, *_: (0, 0)),
    )
    return pl.pallas_call(
        _ffn_kernel,
        grid_spec=grid_spec,
        out_shape=jax.ShapeDtypeStruct((S, D), jnp.float32),
        compiler_params=pltpu.CompilerParams(
            dimension_semantics=("arbitrary", "arbitrary"),
            vmem_limit_bytes=100 * 1024 * 1024),
    )(st, nt, off, xs, W1b, W2b, b1, b2, gamma, beta)


def kernel(x, W1, b1, W2, b2, gamma, beta, orig_input, hash_bin_map):
    flat = x.reshape(S, D)
    bins = jnp.take(hash_bin_map, orig_input.reshape(-1), axis=0)
    oh = (bins[:, None] == jnp.arange(H, dtype=bins.dtype)[None, :])
    oh = oh.astype(jnp.int32)
    csum = jnp.cumsum(oh, axis=0)
    rank = jnp.sum((csum - oh) * oh, axis=1)
    counts = csum[-1]
    off = jnp.concatenate([jnp.zeros((1,), jnp.int32),
                           jnp.cumsum(counts)]).astype(jnp.int32)
    pos = (jnp.take(off, bins) + rank).astype(jnp.int32)
    sort_idx = jnp.zeros((S,), jnp.int32).at[pos].set(
        jnp.arange(S, dtype=jnp.int32))
    xs = jnp.take(flat, sort_idx, axis=0).astype(jnp.bfloat16)
    st = ((off[:H] // 8) * 8).astype(jnp.int32)
    nt = jnp.where(counts > 0,
                   (off[1:] - st + TCHUNK - 1) // TCHUNK,
                   0).astype(jnp.int32)

    out_sorted = _grouped_ffn(
        st, nt, off, xs, W1, W2,
        b1, b2, gamma.reshape(1, D), beta.reshape(1, D))

    out = jnp.take(out_sorted, pos, axis=0)
    return out.reshape(x.shape)
